# baseline (speedup 1.0000x reference)
import functools

import jax
import jax.numpy as jnp
from jax import lax
from jax.experimental import pallas as pl
from jax.experimental.pallas import tpu as pltpu
from jax.experimental.pallas import tpu_sc as plsc

_NC = 2
_NS = 16
_NW = _NC * _NS

_USER_DIM = 64
_GENDER_DIM = 8
_AGE_DIM = 16
_OCC_DIM = 32
_FINAL = 128


def _sc_gather(sample, map_gender, map_age, map_occupation,
               user_pairs, gender_pad, age_pad, occ_pad):
    batch = sample.shape[0]
    bpw = batch // _NW

    mesh = plsc.VectorSubcoreMesh(core_axis_name="c", subcore_axis_name="s")
    out_type = (
        jax.ShapeDtypeStruct((batch, _FINAL), jnp.float32),
        jax.ShapeDtypeStruct((batch, _FINAL), jnp.float32),
        jax.ShapeDtypeStruct((batch, _FINAL), jnp.float32),
        jax.ShapeDtypeStruct((batch, _FINAL), jnp.float32),
    )

    @functools.partial(
        pl.kernel,
        out_type=out_type,
        mesh=mesh,
        scratch_types=[
            pltpu.VMEM((bpw,), jnp.int32),
            pltpu.VMEM((bpw,), jnp.int32),
            pltpu.VMEM((bpw,), jnp.int32),
            pltpu.VMEM((bpw,), jnp.int32),
            pltpu.VMEM((bpw,), jnp.int32),
            pltpu.VMEM((bpw, _FINAL), jnp.float32),
            pltpu.VMEM((bpw, _FINAL), jnp.float32),
            pltpu.VMEM((bpw, _FINAL), jnp.float32),
            pltpu.VMEM((bpw, _FINAL), jnp.float32),
            pltpu.SemaphoreType.DMA,
            pltpu.SemaphoreType.DMA,
            pltpu.SemaphoreType.DMA,
            pltpu.SemaphoreType.DMA,
        ],
    )
    def gather_kernel(sample_h, mg_h, ma_h, mo_h, up_h, ge_h, ae_h, oe_h,
                      fu_o, fg_o, fa_o, fo_o,
                      idx_v, pair_v, gid_v, aid_v, oid_v,
                      fu_v, fg_v, fa_v, fo_v,
                      sem_u, sem_g, sem_a, sem_o):
        wid = lax.axis_index("s") * _NC + lax.axis_index("c")
        base = wid * bpw
        pltpu.sync_copy(sample_h.at[pl.ds(base, bpw)], idx_v)
        for g in range(bpw // 16):
            v = idx_v[pl.ds(g * 16, 16)]
            pair_v[pl.ds(g * 16, 16)] = lax.shift_right_logical(v, 1)
        cp_u = pltpu.async_copy(up_h.at[pair_v], fu_v, sem_u)
        cp_g = pltpu.async_copy(mg_h.at[idx_v], gid_v, sem_g)
        cp_a = pltpu.async_copy(ma_h.at[idx_v], aid_v, sem_a)
        cp_o = pltpu.async_copy(mo_h.at[idx_v], oid_v, sem_o)
        cp_g.wait()
        cp_g2 = pltpu.async_copy(ge_h.at[gid_v], fg_v, sem_g)
        cp_a.wait()
        cp_a2 = pltpu.async_copy(ae_h.at[aid_v], fa_v, sem_a)
        cp_o.wait()
        cp_o2 = pltpu.async_copy(oe_h.at[oid_v], fo_v, sem_o)
        cp_u.wait()
        pltpu.sync_copy(fu_v, fu_o.at[pl.ds(base, bpw)])
        cp_g2.wait()
        pltpu.sync_copy(fg_v, fg_o.at[pl.ds(base, bpw)])
        cp_a2.wait()
        pltpu.sync_copy(fa_v, fa_o.at[pl.ds(base, bpw)])
        cp_o2.wait()
        pltpu.sync_copy(fo_v, fo_o.at[pl.ds(base, bpw)])

    return gather_kernel(sample, map_gender, map_age, map_occupation,
                         user_pairs, gender_pad, age_pad, occ_pad)


def _tc_mlp(fu2, fg, fa, fo, sample, W, b):
    batch = fu2.shape[0]
    bm = 1024

    def body(fu2_r, fg_r, fa_r, fo_r, s_r, w_r, b_r, o_r):
        w = w_r[...]
        u2 = fu2_r[...]
        par = (s_r[...] & 1).reshape(bm, 1)
        u = jnp.where(par == 1, u2[:, _USER_DIM:], u2[:, :_USER_DIM])
        acc = jnp.dot(u, w[0:64], preferred_element_type=jnp.float32)
        acc += jnp.dot(fg_r[...][:, :_GENDER_DIM], w[64:72],
                       preferred_element_type=jnp.float32)
        acc += jnp.dot(fa_r[...][:, :_AGE_DIM], w[72:88],
                       preferred_element_type=jnp.float32)
        acc += jnp.dot(fo_r[...][:, :_OCC_DIM], w[88:120],
                       preferred_element_type=jnp.float32)
        o_r[...] = jnp.maximum(acc + b_r[...].reshape(1, _FINAL), 0.0)

    return pl.pallas_call(
        body,
        grid=(batch // bm,),
        in_specs=[
            pl.BlockSpec((bm, _FINAL), lambda i: (i, 0)),
            pl.BlockSpec((bm, _FINAL), lambda i: (i, 0)),
            pl.BlockSpec((bm, _FINAL), lambda i: (i, 0)),
            pl.BlockSpec((bm, _FINAL), lambda i: (i, 0)),
            pl.BlockSpec((bm,), lambda i: (i,)),
            pl.BlockSpec((120, _FINAL), lambda i: (0, 0)),
            pl.BlockSpec((_FINAL,), lambda i: (0,)),
        ],
        out_specs=pl.BlockSpec((bm, _FINAL), lambda i: (i, 0)),
        out_shape=jax.ShapeDtypeStruct((batch, _FINAL), jnp.float32),
    )(fu2, fg, fa, fo, sample, W, b)


def kernel(sample, map_gender, map_age, map_occupation, user_id_emb,
           gender_emb, age_emb, occupation_emb, W, b):
    user_pairs = user_id_emb.reshape(user_id_emb.shape[0] // 2, 2 * _USER_DIM)
    gender_pad = jnp.pad(gender_emb, ((0, 0), (0, _FINAL - _GENDER_DIM)))
    age_pad = jnp.pad(age_emb, ((0, 0), (0, _FINAL - _AGE_DIM)))
    occ_pad = jnp.pad(occupation_emb, ((0, 0), (0, _FINAL - _OCC_DIM)))
    fu2, fg, fa, fo = _sc_gather(sample, map_gender, map_age, map_occupation,
                                 user_pairs, gender_pad, age_pad, occ_pad)
    return _tc_mlp(fu2, fg, fa, fo, sample, W, b)


# speedup vs baseline: 1.7442x; 1.7442x over previous
import functools

import jax
import jax.numpy as jnp
from jax import lax
from jax.experimental import pallas as pl
from jax.experimental.pallas import tpu as pltpu
from jax.experimental.pallas import tpu_sc as plsc

_NC = 2
_NS = 16
_NW = _NC * _NS

_USER_DIM = 64
_GENDER_DIM = 8
_AGE_DIM = 16
_OCC_DIM = 32
_ATTR_DIM = _GENDER_DIM + _AGE_DIM + _OCC_DIM
_IN_SIZE = _USER_DIM + _ATTR_DIM
_FINAL = 128


def _sc_gather_user(sample, user_id_emb):
    batch = sample.shape[0]
    bpw = batch // _NW

    mesh = plsc.VectorSubcoreMesh(core_axis_name="c", subcore_axis_name="s")

    @functools.partial(
        pl.kernel,
        out_type=jax.ShapeDtypeStruct((batch, _USER_DIM), jnp.float32),
        mesh=mesh,
        compiler_params=pltpu.CompilerParams(use_tc_tiling_on_sc=True),
        scratch_types=[
            pltpu.VMEM((bpw,), jnp.int32),
            pltpu.VMEM((bpw, _USER_DIM), jnp.float32),
            pltpu.SemaphoreType.DMA,
        ],
    )
    def user_kernel(sample_h, ue_h, out_h, idx_v, fu_v, sem_u):
        wid = lax.axis_index("s") * _NC + lax.axis_index("c")
        base = wid * bpw
        pltpu.sync_copy(sample_h.at[pl.ds(base, bpw)], idx_v)
        cps = []
        for g in range(bpw // 16):
            vg = idx_v[pl.ds(g * 16, 16)]
            for j in range(16):
                r = vg[j]
                cps.append(pltpu.async_copy(
                    ue_h.at[pl.ds(r, 1), :],
                    fu_v.at[pl.ds(g * 16 + j, 1), :],
                    sem_u))
        for cp in cps:
            cp.wait()
        pltpu.sync_copy(fu_v, out_h.at[pl.ds(base, bpw), :])

    return user_kernel(sample, user_id_emb)


def _sc_gather_attrs(sample, map_gender, map_age, map_occupation,
                     gender_emb, age_emb, occupation_emb):
    batch = sample.shape[0]
    bpw = batch // _NW

    mesh = plsc.VectorSubcoreMesh(core_axis_name="c", subcore_axis_name="s")

    @functools.partial(
        pl.kernel,
        out_type=jax.ShapeDtypeStruct((batch, _FINAL), jnp.float32),
        mesh=mesh,
        compiler_params=pltpu.CompilerParams(use_tc_tiling_on_sc=False),
        scratch_types=[
            pltpu.VMEM((bpw,), jnp.int32),
            pltpu.VMEM((bpw,), jnp.int32),
            pltpu.VMEM((bpw,), jnp.int32),
            pltpu.VMEM((bpw,), jnp.int32),
            pltpu.VMEM((bpw, _GENDER_DIM), jnp.float32),
            pltpu.VMEM((bpw, _AGE_DIM), jnp.float32),
            pltpu.VMEM((bpw, _OCC_DIM), jnp.float32),
            pltpu.SemaphoreType.DMA,
            pltpu.SemaphoreType.DMA,
            pltpu.SemaphoreType.DMA,
        ],
    )
    def attr_kernel(sample_h, mg_h, ma_h, mo_h, ge_h, ae_h, oe_h,
                    feats_o,
                    idx_v, gid_v, aid_v, oid_v, fg_v, fa_v, fo_v,
                    sem_g, sem_a, sem_o):
        wid = lax.axis_index("s") * _NC + lax.axis_index("c")
        base = wid * bpw
        pltpu.sync_copy(sample_h.at[pl.ds(base, bpw)], idx_v)
        cp_g = pltpu.async_copy(mg_h.at[idx_v], gid_v, sem_g)
        cp_a = pltpu.async_copy(ma_h.at[idx_v], aid_v, sem_a)
        cp_o = pltpu.async_copy(mo_h.at[idx_v], oid_v, sem_o)
        cp_g.wait()
        cp_g2 = pltpu.async_copy(ge_h.at[gid_v], fg_v, sem_g)
        cp_a.wait()
        cp_a2 = pltpu.async_copy(ae_h.at[aid_v], fa_v, sem_a)
        cp_o.wait()
        cp_o2 = pltpu.async_copy(oe_h.at[oid_v], fo_v, sem_o)
        cp_g2.wait()
        pltpu.sync_copy(
            fg_v, feats_o.at[pl.ds(base, bpw), pl.ds(0, _GENDER_DIM)])
        cp_a2.wait()
        pltpu.sync_copy(
            fa_v, feats_o.at[pl.ds(base, bpw), pl.ds(_GENDER_DIM, _AGE_DIM)])
        cp_o2.wait()
        pltpu.sync_copy(
            fo_v, feats_o.at[pl.ds(base, bpw), pl.ds(24, _OCC_DIM)])

    return attr_kernel(sample, map_gender, map_age, map_occupation,
                       gender_emb, age_emb, occupation_emb)


def _tc_mlp(f_user, f_attr, W, b):
    batch = f_user.shape[0]
    bm = 1024

    def body(fu_r, fa_r, w_r, b_r, o_r):
        xu = fu_r[...]
        xa = fa_r[...][:, :_ATTR_DIM]
        w = w_r[...]
        acc = jnp.dot(xu, w[:_USER_DIM], preferred_element_type=jnp.float32)
        acc += jnp.dot(xa, w[_USER_DIM:_IN_SIZE],
                       preferred_element_type=jnp.float32)
        o_r[...] = jnp.maximum(acc + b_r[...].reshape(1, _FINAL), 0.0)

    return pl.pallas_call(
        body,
        grid=(batch // bm,),
        in_specs=[
            pl.BlockSpec((bm, _USER_DIM), lambda i: (i, 0)),
            pl.BlockSpec((bm, _FINAL), lambda i: (i, 0)),
            pl.BlockSpec((_IN_SIZE, _FINAL), lambda i: (0, 0)),
            pl.BlockSpec((_FINAL,), lambda i: (0,)),
        ],
        out_specs=pl.BlockSpec((bm, _FINAL), lambda i: (i, 0)),
        out_shape=jax.ShapeDtypeStruct((batch, _FINAL), jnp.float32),
    )(f_user, f_attr, W, b)


def kernel(sample, map_gender, map_age, map_occupation, user_id_emb,
           gender_emb, age_emb, occupation_emb, W, b):
    f_user = _sc_gather_user(sample, user_id_emb)
    f_attr = _sc_gather_attrs(sample, map_gender, map_age, map_occupation,
                              gender_emb, age_emb, occupation_emb)
    return _tc_mlp(f_user, f_attr, W, b)


# speedup vs baseline: 1.7513x; 1.0041x over previous
import functools

import jax
import jax.numpy as jnp
from jax import lax
from jax.experimental import pallas as pl
from jax.experimental.pallas import tpu as pltpu
from jax.experimental.pallas import tpu_sc as plsc

_NC = 2
_NS = 16
_NW = _NC * _NS

_USER_DIM = 64
_GENDER_DIM = 8
_AGE_DIM = 16
_OCC_DIM = 32
_ATTR_DIM = _GENDER_DIM + _AGE_DIM + _OCC_DIM
_IN_SIZE = _USER_DIM + _ATTR_DIM
_FINAL = 128


def _sc_gather_user(sample, user_id_emb):
    batch = sample.shape[0]
    bpw = batch // _NW

    mesh = plsc.VectorSubcoreMesh(core_axis_name="c", subcore_axis_name="s")

    @functools.partial(
        pl.kernel,
        out_type=jax.ShapeDtypeStruct((batch, _USER_DIM), jnp.float32),
        mesh=mesh,
        compiler_params=pltpu.CompilerParams(use_tc_tiling_on_sc=True),
        scratch_types=[
            pltpu.VMEM((bpw,), jnp.int32),
            pltpu.VMEM((bpw, _USER_DIM), jnp.float32),
            pltpu.SemaphoreType.DMA,
        ],
    )
    def user_kernel(sample_h, ue_h, out_h, idx_v, fu_v, sem_u):
        wid = lax.axis_index("s") * _NC + lax.axis_index("c")
        base = wid * bpw
        pltpu.sync_copy(sample_h.at[pl.ds(base, bpw)], idx_v)
        cps = []
        for g in range(bpw // 16):
            vg = idx_v[pl.ds(g * 16, 16)]
            for j in range(16):
                r = vg[j]
                cps.append(pltpu.async_copy(
                    ue_h.at[pl.ds(r, 1), :],
                    fu_v.at[pl.ds(g * 16 + j, 1), :],
                    sem_u))
        for cp in cps:
            cp.wait()
        pltpu.sync_copy(fu_v, out_h.at[pl.ds(base, bpw), :])

    return user_kernel(sample, user_id_emb)


def _sc_gather_attrs(sample, map_gender, map_age, map_occupation,
                     gender_emb, age_emb, occupation_emb):
    batch = sample.shape[0]
    bpw = batch // _NW

    mesh = plsc.VectorSubcoreMesh(core_axis_name="c", subcore_axis_name="s")

    @functools.partial(
        pl.kernel,
        out_type=jax.ShapeDtypeStruct((batch, _FINAL), jnp.float32),
        mesh=mesh,
        compiler_params=pltpu.CompilerParams(use_tc_tiling_on_sc=False),
        scratch_types=[
            pltpu.VMEM((bpw,), jnp.int32),
            pltpu.VMEM((bpw,), jnp.int32),
            pltpu.VMEM((bpw,), jnp.int32),
            pltpu.VMEM((bpw,), jnp.int32),
            pltpu.VMEM((bpw, _GENDER_DIM), jnp.float32),
            pltpu.VMEM((bpw, _AGE_DIM), jnp.float32),
            pltpu.VMEM((bpw, _OCC_DIM), jnp.float32),
            pltpu.SemaphoreType.DMA,
            pltpu.SemaphoreType.DMA,
            pltpu.SemaphoreType.DMA,
        ],
    )
    def attr_kernel(sample_h, mg_h, ma_h, mo_h, ge_h, ae_h, oe_h,
                    feats_o,
                    idx_v, gid_v, aid_v, oid_v, fg_v, fa_v, fo_v,
                    sem_g, sem_a, sem_o):
        wid = lax.axis_index("s") * _NC + lax.axis_index("c")
        base = wid * bpw
        pltpu.sync_copy(sample_h.at[pl.ds(base, bpw)], idx_v)
        cp_g = pltpu.async_copy(mg_h.at[idx_v], gid_v, sem_g)
        cp_a = pltpu.async_copy(ma_h.at[idx_v], aid_v, sem_a)
        cp_o = pltpu.async_copy(mo_h.at[idx_v], oid_v, sem_o)
        cp_g.wait()
        cp_g2 = pltpu.async_copy(ge_h.at[gid_v], fg_v, sem_g)
        cp_a.wait()
        cp_a2 = pltpu.async_copy(ae_h.at[aid_v], fa_v, sem_a)
        cp_o.wait()
        cp_o2 = pltpu.async_copy(oe_h.at[oid_v], fo_v, sem_o)
        cp_g2.wait()
        wr_g = pltpu.async_copy(
            fg_v, feats_o.at[pl.ds(base, bpw), pl.ds(0, _GENDER_DIM)], sem_g)
        cp_a2.wait()
        wr_a = pltpu.async_copy(
            fa_v, feats_o.at[pl.ds(base, bpw), pl.ds(_GENDER_DIM, _AGE_DIM)],
            sem_a)
        cp_o2.wait()
        wr_o = pltpu.async_copy(
            fo_v, feats_o.at[pl.ds(base, bpw), pl.ds(24, _OCC_DIM)], sem_o)
        wr_g.wait()
        wr_a.wait()
        wr_o.wait()

    return attr_kernel(sample, map_gender, map_age, map_occupation,
                       gender_emb, age_emb, occupation_emb)


def _tc_mlp(f_user, f_attr, W, b):
    batch = f_user.shape[0]
    bm = 1024

    def body(fu_r, fa_r, w_r, b_r, o_r):
        xu = fu_r[...]
        xa = fa_r[...][:, :_ATTR_DIM]
        w = w_r[...]
        acc = jnp.dot(xu, w[:_USER_DIM], preferred_element_type=jnp.float32)
        acc += jnp.dot(xa, w[_USER_DIM:_IN_SIZE],
                       preferred_element_type=jnp.float32)
        o_r[...] = jnp.maximum(acc + b_r[...].reshape(1, _FINAL), 0.0)

    return pl.pallas_call(
        body,
        grid=(batch // bm,),
        in_specs=[
            pl.BlockSpec((bm, _USER_DIM), lambda i: (i, 0)),
            pl.BlockSpec((bm, _FINAL), lambda i: (i, 0)),
            pl.BlockSpec((_IN_SIZE, _FINAL), lambda i: (0, 0)),
            pl.BlockSpec((_FINAL,), lambda i: (0,)),
        ],
        out_specs=pl.BlockSpec((bm, _FINAL), lambda i: (i, 0)),
        out_shape=jax.ShapeDtypeStruct((batch, _FINAL), jnp.float32),
    )(f_user, f_attr, W, b)


def kernel(sample, map_gender, map_age, map_occupation, user_id_emb,
           gender_emb, age_emb, occupation_emb, W, b):
    f_user = _sc_gather_user(sample, user_id_emb)
    f_attr = _sc_gather_attrs(sample, map_gender, map_age, map_occupation,
                              gender_emb, age_emb, occupation_emb)
    return _tc_mlp(f_user, f_attr, W, b)
